# baseline (speedup 1.0000x reference)
import jax
import jax.numpy as jnp
from jax import lax
from jax.experimental import pallas as pl
from jax.experimental.pallas import tpu as pltpu
from jax.experimental.pallas import tpu_sc as plsc

N = 20000
NUM_TILES = 32
ROWS_PER_TILE = 640
NUM_GROUPS = ROWS_PER_TILE // 16
MAX_BASE_LIN = N - ROWS_PER_TILE
MAX_BASE_2D = 19456
BETA = 1.0 / 9.0

_LOG1P_C = (
    1.693662625257275e-06, 0.9998325705528259, -0.4972033202648163,
    0.31504127383232117, -0.18901954591274261, 0.08152318000793457,
    -0.01702961139380932,
)


def _tile_body(logits_hbm, er_hbm, tgt_hbm, lab_hbm, out_hbm,
               l01_v, er_v, tgt_v, lab_v, acc_v, sem):
    c = lax.axis_index("c")
    s = lax.axis_index("s")
    gid = s * 2 + c
    nominal = gid * ROWS_PER_TILE
    base = jnp.minimum(nominal, MAX_BASE_LIN)
    off = nominal - base
    base2 = jnp.minimum(nominal, MAX_BASE_2D)
    off2 = nominal - base2

    R = ROWS_PER_TILE
    cps_ce = [
        pltpu.async_copy(logits_hbm.at[:, pl.ds(base2, R)], l01_v, sem),
        pltpu.async_copy(lab_hbm.at[pl.ds(base, R)], lab_v, sem),
    ]
    cps_reg = [
        pltpu.async_copy(er_hbm.at[:, pl.ds(base2, R)], er_v, sem),
        pltpu.async_copy(tgt_hbm.at[:, pl.ds(base2, R)], tgt_v, sem),
    ]
    for cp in cps_ce:
        cp.wait()

    inv_n = jnp.float32(1.0 / N)

    def ce_group(g, carry):
        valid = nominal + g * 16 < N
        lrow = jnp.minimum(off + g * 16, ROWS_PER_TILE - 16)
        lrow2 = jnp.minimum(off2 + g * 16, ROWS_PER_TILE - 16)
        lab = lab_v[pl.ds(lrow, 16)]
        labf = lab.astype(jnp.float32)
        l0 = l01_v[0, pl.ds(lrow2, 16)]
        l1 = l01_v[1, pl.ds(lrow2, 16)]
        gm = (l1 - l0) * (1.0 - 2.0 * labf)
        t = jnp.exp(-jnp.abs(gm))
        p = jnp.float32(_LOG1P_C[6])
        for ck in _LOG1P_C[5::-1]:
            p = p * t + jnp.float32(ck)
        ce = jnp.maximum(gm, 0.0) + p
        return carry + jnp.where(valid, ce, 0.0)

    acc_ce = lax.fori_loop(0, NUM_GROUPS, ce_group,
                           jnp.zeros((16,), jnp.float32), unroll=8)

    for cp in cps_reg:
        cp.wait()

    def reg_group(g, carry):
        valid = nominal + g * 16 < N
        lrow = jnp.minimum(off + g * 16, ROWS_PER_TILE - 16)
        lrow2 = jnp.minimum(off2 + g * 16, ROWS_PER_TILE - 16)
        labf = lab_v[pl.ds(lrow, 16)].astype(jnp.float32)
        sl_sum = jnp.zeros((16,), jnp.float32)
        for j in range(6):
            er = er_v[6 + j, pl.ds(lrow2, 16)]
            tgt = tgt_v[j, pl.ds(lrow2, 16)]
            d = er - tgt
            a = jnp.abs(d)
            sl_sum = sl_sum + jnp.where(
                a < BETA, (0.5 / BETA) * d * d, a - 0.5 * BETA)
        return carry + jnp.where(valid, sl_sum * labf, 0.0)

    acc_sl = lax.fori_loop(0, NUM_GROUPS, reg_group,
                           jnp.zeros((16,), jnp.float32), unroll=8)

    acc_v[...] = acc_ce * inv_n
    pltpu.sync_copy(acc_v, out_hbm.at[pl.ds(gid * 16, 16)])
    acc_v[...] = acc_sl * inv_n
    pltpu.sync_copy(acc_v, out_hbm.at[pl.ds(512 + gid * 16, 16)])


_sc_call = pl.kernel(
    _tile_body,
    out_type=jax.ShapeDtypeStruct((1024,), jnp.float32),
    mesh=plsc.VectorSubcoreMesh(core_axis_name="c", subcore_axis_name="s"),
    compiler_params=pltpu.CompilerParams(
        needs_layout_passes=False, use_tc_tiling_on_sc=True,
        skip_device_barrier=True),
    scratch_types=[
        pltpu.VMEM((2, ROWS_PER_TILE), jnp.float32),
        pltpu.VMEM((12, ROWS_PER_TILE), jnp.float32),
        pltpu.VMEM((6, ROWS_PER_TILE), jnp.float32),
        pltpu.VMEM((ROWS_PER_TILE,), jnp.int32),
        pltpu.VMEM((16,), jnp.float32),
        pltpu.SemaphoreType.DMA,
    ],
)


@jax.jit
def kernel(class_logits, ellipse_regression, labels_cat, regression_targets):
    parts = _sc_call(
        class_logits.T,
        ellipse_regression.T,
        regression_targets.T,
        labels_cat.astype(jnp.int32),
    )
    return jnp.sum(parts[:512]), jnp.sum(parts[512:])


# speedup vs baseline: 1.1709x; 1.1709x over previous
import jax
import jax.numpy as jnp
from jax import lax
from jax.experimental import pallas as pl
from jax.experimental.pallas import tpu as pltpu
from jax.experimental.pallas import tpu_sc as plsc

N = 20000
NUM_TILES = 32
ROWS_PER_TILE = 640
NUM_GROUPS = ROWS_PER_TILE // 16
MAX_BASE_LIN = N - ROWS_PER_TILE
MAX_BASE_2D = 19456
BETA = 1.0 / 9.0

_LOG1P_C = (
    1.693662625257275e-06, 0.9998325705528259, -0.4972033202648163,
    0.31504127383232117, -0.18901954591274261, 0.08152318000793457,
    -0.01702961139380932,
)


def _tile_body(logits_hbm, er_hbm, tgt_hbm, lab_hbm, out_hbm,
               l01_v, er_v, tgt_v, lab_v, acc_v, sem):
    c = lax.axis_index("c")
    s = lax.axis_index("s")
    gid = s * 2 + c
    nominal = gid * ROWS_PER_TILE
    base = jnp.minimum(nominal, MAX_BASE_LIN)
    off = nominal - base
    base2 = jnp.minimum(nominal, MAX_BASE_2D)
    off2 = nominal - base2

    R = ROWS_PER_TILE
    cps_ce = [
        pltpu.async_copy(logits_hbm.at[:, pl.ds(base2, R)], l01_v, sem),
        pltpu.async_copy(lab_hbm.at[pl.ds(base, R)], lab_v, sem),
    ]
    cps_reg = [
        pltpu.async_copy(er_hbm.at[:, pl.ds(base2, R)], er_v, sem),
        pltpu.async_copy(tgt_hbm.at[:, pl.ds(base2, R)], tgt_v, sem),
    ]
    for cp in cps_ce:
        cp.wait()

    inv_n = jnp.float32(1.0 / N)

    def ce_group(g, carry):
        valid = nominal + g * 16 < N
        lrow = jnp.minimum(off + g * 16, ROWS_PER_TILE - 16)
        lrow2 = jnp.minimum(off2 + g * 16, ROWS_PER_TILE - 16)
        lab = lab_v[pl.ds(lrow, 16)]
        labf = lab.astype(jnp.float32)
        l0 = l01_v[0, pl.ds(lrow2, 16)]
        l1 = l01_v[1, pl.ds(lrow2, 16)]
        gm = (l1 - l0) * (1.0 - 2.0 * labf)
        t = jnp.exp(-jnp.abs(gm))
        p = jnp.float32(_LOG1P_C[6])
        for ck in _LOG1P_C[5::-1]:
            p = p * t + jnp.float32(ck)
        ce = jnp.maximum(gm, 0.0) + p
        return carry + jnp.where(valid, ce, 0.0)

    acc_ce = lax.fori_loop(0, NUM_GROUPS, ce_group,
                           jnp.zeros((16,), jnp.float32), unroll=2)

    for cp in cps_reg:
        cp.wait()

    def reg_group(g, carry):
        valid = nominal + g * 16 < N
        lrow = jnp.minimum(off + g * 16, ROWS_PER_TILE - 16)
        lrow2 = jnp.minimum(off2 + g * 16, ROWS_PER_TILE - 16)
        labf = lab_v[pl.ds(lrow, 16)].astype(jnp.float32)
        sl_sum = jnp.zeros((16,), jnp.float32)
        for j in range(6):
            er = er_v[6 + j, pl.ds(lrow2, 16)]
            tgt = tgt_v[j, pl.ds(lrow2, 16)]
            d = er - tgt
            a = jnp.abs(d)
            sl_sum = sl_sum + jnp.where(
                a < BETA, (0.5 / BETA) * d * d, a - 0.5 * BETA)
        return carry + jnp.where(valid, sl_sum * labf, 0.0)

    acc_sl = lax.fori_loop(0, NUM_GROUPS, reg_group,
                           jnp.zeros((16,), jnp.float32), unroll=2)

    acc_v[...] = acc_ce * inv_n
    pltpu.sync_copy(acc_v, out_hbm.at[pl.ds(gid * 16, 16)])
    acc_v[...] = acc_sl * inv_n
    pltpu.sync_copy(acc_v, out_hbm.at[pl.ds(512 + gid * 16, 16)])


_sc_call = pl.kernel(
    _tile_body,
    out_type=jax.ShapeDtypeStruct((1024,), jnp.float32),
    mesh=plsc.VectorSubcoreMesh(core_axis_name="c", subcore_axis_name="s"),
    compiler_params=pltpu.CompilerParams(
        needs_layout_passes=False, use_tc_tiling_on_sc=True,
        skip_device_barrier=True),
    scratch_types=[
        pltpu.VMEM((2, ROWS_PER_TILE), jnp.float32),
        pltpu.VMEM((12, ROWS_PER_TILE), jnp.float32),
        pltpu.VMEM((6, ROWS_PER_TILE), jnp.float32),
        pltpu.VMEM((ROWS_PER_TILE,), jnp.int32),
        pltpu.VMEM((16,), jnp.float32),
        pltpu.SemaphoreType.DMA,
    ],
)


@jax.jit
def kernel(class_logits, ellipse_regression, labels_cat, regression_targets):
    parts = _sc_call(
        class_logits.T,
        ellipse_regression.T,
        regression_targets.T,
        labels_cat.astype(jnp.int32),
    )
    return jnp.sum(parts[:512]), jnp.sum(parts[512:])


# speedup vs baseline: 1.1783x; 1.0063x over previous
import jax
import jax.numpy as jnp
from jax import lax
from jax.experimental import pallas as pl
from jax.experimental.pallas import tpu as pltpu
from jax.experimental.pallas import tpu_sc as plsc

N = 20000
NUM_TILES = 32
ROWS_PER_TILE = 640
NUM_GROUPS = ROWS_PER_TILE // 16
MAX_BASE_LIN = N - ROWS_PER_TILE
MAX_BASE_2D = 19456
BETA = 1.0 / 9.0

_LOG1P_C = (
    1.693662625257275e-06, 0.9998325705528259, -0.4972033202648163,
    0.31504127383232117, -0.18901954591274261, 0.08152318000793457,
    -0.01702961139380932,
)


def _tile_body(logits_hbm, er_hbm, tgt_hbm, lab_hbm, out_hbm,
               l01_v, er_v, tgt_v, lab_v, acc_v, sem):
    c = lax.axis_index("c")
    s = lax.axis_index("s")
    gid = s * 2 + c
    nominal = gid * ROWS_PER_TILE
    base = jnp.minimum(nominal, MAX_BASE_LIN)
    off = nominal - base
    base2 = jnp.minimum(nominal, MAX_BASE_2D)
    off2 = nominal - base2

    R = ROWS_PER_TILE
    cps_ce = [
        pltpu.async_copy(logits_hbm.at[:, pl.ds(base2, R)], l01_v, sem),
        pltpu.async_copy(lab_hbm.at[pl.ds(base, R)], lab_v, sem),
    ]
    cps_reg = [
        pltpu.async_copy(er_hbm.at[:, pl.ds(base2, R)], er_v, sem),
        pltpu.async_copy(tgt_hbm.at[:, pl.ds(base2, R)], tgt_v, sem),
    ]
    for cp in cps_ce:
        cp.wait()

    inv_n = jnp.float32(1.0 / N)

    def ce_group(g, carry):
        valid = nominal + g * 16 < N
        lrow = jnp.minimum(off + g * 16, ROWS_PER_TILE - 16)
        lrow2 = jnp.minimum(off2 + g * 16, ROWS_PER_TILE - 16)
        lab = lab_v[pl.ds(lrow, 16)]
        labf = lab.astype(jnp.float32)
        l0 = l01_v[0, pl.ds(lrow2, 16)]
        l1 = l01_v[1, pl.ds(lrow2, 16)]
        gm = (l1 - l0) * (1.0 - 2.0 * labf)
        t = jnp.exp(-jnp.abs(gm))
        p = jnp.float32(_LOG1P_C[6])
        for ck in _LOG1P_C[5::-1]:
            p = p * t + jnp.float32(ck)
        ce = jnp.maximum(gm, 0.0) + p
        return carry + jnp.where(valid, ce, 0.0)

    acc_ce = lax.fori_loop(0, NUM_GROUPS, ce_group,
                           jnp.zeros((16,), jnp.float32))

    for cp in cps_reg:
        cp.wait()

    def reg_group(g, carry):
        valid = nominal + g * 16 < N
        lrow = jnp.minimum(off + g * 16, ROWS_PER_TILE - 16)
        lrow2 = jnp.minimum(off2 + g * 16, ROWS_PER_TILE - 16)
        labf = lab_v[pl.ds(lrow, 16)].astype(jnp.float32)
        sl_sum = jnp.zeros((16,), jnp.float32)
        for j in range(6):
            er = er_v[6 + j, pl.ds(lrow2, 16)]
            tgt = tgt_v[j, pl.ds(lrow2, 16)]
            d = er - tgt
            a = jnp.abs(d)
            sl_sum = sl_sum + jnp.where(
                a < BETA, (0.5 / BETA) * d * d, a - 0.5 * BETA)
        return carry + jnp.where(valid, sl_sum * labf, 0.0)

    acc_sl = lax.fori_loop(0, NUM_GROUPS, reg_group,
                           jnp.zeros((16,), jnp.float32))

    acc_v[...] = acc_ce * inv_n
    pltpu.sync_copy(acc_v, out_hbm.at[pl.ds(gid * 16, 16)])
    acc_v[...] = acc_sl * inv_n
    pltpu.sync_copy(acc_v, out_hbm.at[pl.ds(512 + gid * 16, 16)])


_sc_call = pl.kernel(
    _tile_body,
    out_type=jax.ShapeDtypeStruct((1024,), jnp.float32),
    mesh=plsc.VectorSubcoreMesh(core_axis_name="c", subcore_axis_name="s"),
    compiler_params=pltpu.CompilerParams(
        needs_layout_passes=False, use_tc_tiling_on_sc=True,
        skip_device_barrier=True),
    scratch_types=[
        pltpu.VMEM((2, ROWS_PER_TILE), jnp.float32),
        pltpu.VMEM((12, ROWS_PER_TILE), jnp.float32),
        pltpu.VMEM((6, ROWS_PER_TILE), jnp.float32),
        pltpu.VMEM((ROWS_PER_TILE,), jnp.int32),
        pltpu.VMEM((16,), jnp.float32),
        pltpu.SemaphoreType.DMA,
    ],
)


@jax.jit
def kernel(class_logits, ellipse_regression, labels_cat, regression_targets):
    parts = _sc_call(
        class_logits.T,
        ellipse_regression.T,
        regression_targets.T,
        labels_cat.astype(jnp.int32),
    )
    return jnp.sum(parts[:512]), jnp.sum(parts[512:])
